# baseline (speedup 1.0000x reference)
import functools

import jax
import jax.numpy as jnp
from jax import lax
from jax.experimental import pallas as pl
from jax.experimental.pallas import tpu as pltpu
from jax.experimental.pallas import tpu_sc as plsc

_NC = 2
_NS = 16
_NW = _NC * _NS

_D = 32
_G = 128
_W = 128
_TP = 130


def _gather_body(idxT_hbm, table4_hbm, out_hbm, idx_v, idx4_v,
                 r0, r1, t0, t1, sg0, sg1, so0, so1):
    H = idxT_hbm.shape[0]
    B = idxT_hbm.shape[1]
    bw = B // _NW
    jb = bw // _G
    nblk = H * jb
    wid = lax.axis_index("s") * _NC + lax.axis_index("c")
    bbase = wid * bw

    for j in range(jb):
        pltpu.sync_copy(idxT_hbm.at[:, pl.ds(bbase + j * _G, _G)],
                        idx_v.at[j])

    def sbody(k, carry):
        j = k // (H * 8)
        rem = k % (H * 8)
        h = rem // 8
        g = rem % 8
        v = idx_v[j, h, pl.ds(g * 16, 16)]
        idx4_v[j, h, pl.ds(g * 16, 16)] = jax.lax.shift_right_logical(v, 2)
        return carry

    lax.fori_loop(0, jb * H * 8, sbody, 0, unroll=8)

    rows = (r0, r1)
    trs = (t0, t1)
    sg = (sg0, sg1)
    so = (so0, so1)
    iota = lax.iota(jnp.int32, 16)
    bidxs = [iota + 16 * g for g in range(8)]

    def issue_gather(i, p):
        pltpu.async_copy(table4_hbm.at[idx4_v.at[i % jb, i // jb]],
                         rows[p], sg[p])

    def drain_gather(p):
        pltpu.make_async_copy(table4_hbm.at[pl.ds(0, _G)], rows[p],
                              sg[p]).wait()

    def transpose(i, p):
        r, t = rows[p], trs[p]
        jj = i % jb
        hh = i // jb
        mcols = [(idx_v[jj, hh, pl.ds(g * 16, 16)] & 3) * 32
                 for g in range(8)]

        def tbody(d, carry):
            dvec = (d + iota) & 31
            for g in range(8):
                v = plsc.load_gather(r, [bidxs[g], mcols[g] + dvec])
                plsc.store_scatter(t, [dvec, bidxs[g]], v)
            return carry

        lax.fori_loop(0, _D, tbody, 0, unroll=2)

    def issue_store(i, p):
        pltpu.async_copy(
            trs[p].at[:, pl.ds(0, _G)],
            out_hbm.at[i // jb, :, pl.ds(bbase + (i % jb) * _G, _G)],
            so[p])

    def drain_store(p):
        pltpu.make_async_copy(trs[p].at[:, pl.ds(0, _G)],
                              out_hbm.at[0, :, pl.ds(0, _G)], so[p]).wait()

    issue_gather(0, 0)
    issue_gather(1, 1)
    drain_gather(0)
    transpose(0, 0)
    issue_store(0, 0)

    def body(g, carry):
        i0 = 2 * g
        drain_store(0)
        issue_gather(i0, 0)
        drain_gather(1)
        transpose(i0 - 1, 1)
        issue_store(i0 - 1, 1)
        drain_store(1)
        issue_gather(i0 + 1, 1)
        drain_gather(0)
        transpose(i0, 0)
        issue_store(i0, 0)
        return carry

    lax.fori_loop(1, nblk // 2, body, 0)

    drain_gather(1)
    transpose(nblk - 1, 1)
    issue_store(nblk - 1, 1)
    drain_store(0)
    drain_store(1)


def kernel(inputs, table):
    batch, hist = inputs.shape
    nvocab, dim = table.shape
    bw = batch // _NW
    jb = bw // _G

    mesh = plsc.VectorSubcoreMesh(core_axis_name="c", subcore_axis_name="s")
    run = functools.partial(
        pl.kernel,
        mesh=mesh,
        compiler_params=pltpu.CompilerParams(
            use_tc_tiling_on_sc=False, needs_layout_passes=False),
        out_type=jax.ShapeDtypeStruct((hist, _D, batch), jnp.float32),
        scratch_types=(
            [pltpu.VMEM((jb, hist, _G), jnp.int32) for _ in range(2)]
            + [pltpu.VMEM((_G, _W), jnp.float32) for _ in range(2)]
            + [pltpu.VMEM((_D, _TP), jnp.float32) for _ in range(2)]
            + [pltpu.SemaphoreType.DMA for _ in range(4)]
        ),
    )(_gather_body)

    table4 = table.reshape(nvocab * dim // _W, _W)
    out_phys = run(inputs.T, table4)
    return out_phys.transpose(2, 0, 1)


# speedup vs baseline: 1.1397x; 1.1397x over previous
import functools

import jax
import jax.numpy as jnp
from jax import lax
from jax.experimental import pallas as pl
from jax.experimental.pallas import tpu as pltpu
from jax.experimental.pallas import tpu_sc as plsc

_NC = 2
_NS = 16
_NW = _NC * _NS

_D = 32
_G = 128
_W = 128
_TP = 128


def _gather_body(idxT_hbm, table4_hbm, out_hbm, idx_v, idx4_v,
                 r0, r1, t0, t1, sg0, sg1, so0, so1):
    H = idxT_hbm.shape[0]
    B = idxT_hbm.shape[1]
    bw = B // _NW
    jb = bw // _G
    nblk = H * jb
    wid = lax.axis_index("s") * _NC + lax.axis_index("c")
    bbase = wid * bw

    for j in range(jb):
        pltpu.sync_copy(idxT_hbm.at[:, pl.ds(bbase + j * _G, _G)],
                        idx_v.at[j, pl.ds(0, H)])

    def sbody(k, carry):
        j = k // (H * 8)
        rem = k % (H * 8)
        h = rem // 8
        g = rem % 8
        v = idx_v[j, h, pl.ds(g * 16, 16)]
        idx4_v[j, h, pl.ds(g * 16, 16)] = jax.lax.shift_right_logical(v, 2)
        return carry

    lax.fori_loop(0, jb * H * 8, sbody, 0, unroll=8)

    rows = (r0, r1)
    trs = (t0, t1)
    sg = (sg0, sg1)
    so = (so0, so1)
    iota = lax.iota(jnp.int32, 16)
    bidxs = [iota + 16 * g for g in range(8)]

    def issue_gather(i, p):
        pltpu.async_copy(table4_hbm.at[idx4_v.at[i % jb, i // jb]],
                         rows[p], sg[p])

    def drain_gather(p):
        pltpu.make_async_copy(table4_hbm.at[pl.ds(0, _G)], rows[p],
                              sg[p]).wait()

    def transpose(i, p):
        r, t = rows[p], trs[p]
        jj = i % jb
        hh = i // jb
        mcols = [(idx_v[jj, hh, pl.ds(g * 16, 16)] & 3) * 32
                 for g in range(8)]

        def tbody(d, carry):
            dvec = (d + iota) & 31
            for g in range(8):
                v = plsc.load_gather(r, [bidxs[g], mcols[g] + dvec])
                plsc.store_scatter(t, [dvec, bidxs[g]], v)
            return carry

        lax.fori_loop(0, _D, tbody, 0, unroll=2)

    def issue_store(i, p):
        pltpu.async_copy(
            trs[p].at[:, pl.ds(0, _G)],
            out_hbm.at[i // jb, :, pl.ds(bbase + (i % jb) * _G, _G)],
            so[p])

    def drain_store(p):
        pltpu.make_async_copy(trs[p].at[:, pl.ds(0, _G)],
                              out_hbm.at[0, :, pl.ds(0, _G)], so[p]).wait()

    issue_gather(0, 0)
    issue_gather(1, 1)
    drain_gather(0)
    transpose(0, 0)
    issue_store(0, 0)

    def body(g, carry):
        i0 = 2 * g
        drain_store(0)
        issue_gather(i0, 0)
        drain_gather(1)
        transpose(i0 - 1, 1)
        issue_store(i0 - 1, 1)
        drain_store(1)
        issue_gather(i0 + 1, 1)
        drain_gather(0)
        transpose(i0, 0)
        issue_store(i0, 0)
        return carry

    lax.fori_loop(1, nblk // 2, body, 0)

    drain_gather(1)
    transpose(nblk - 1, 1)
    issue_store(nblk - 1, 1)
    drain_store(0)
    drain_store(1)


def kernel(inputs, table):
    batch, hist = inputs.shape
    nvocab, dim = table.shape
    bw = batch // _NW
    jb = bw // _G

    mesh = plsc.VectorSubcoreMesh(core_axis_name="c", subcore_axis_name="s")
    run = functools.partial(
        pl.kernel,
        mesh=mesh,
        compiler_params=pltpu.CompilerParams(
            use_tc_tiling_on_sc=True, needs_layout_passes=False),
        out_type=jax.ShapeDtypeStruct((hist, _D, batch), jnp.float32),
        scratch_types=(
            [pltpu.VMEM((jb, (hist + 7) // 8 * 8, _G), jnp.int32)
             for _ in range(2)]
            + [pltpu.VMEM((_G, _W), jnp.float32) for _ in range(2)]
            + [pltpu.VMEM((_D, _TP), jnp.float32) for _ in range(2)]
            + [pltpu.SemaphoreType.DMA for _ in range(4)]
        ),
    )(_gather_body)

    table4 = table.reshape(nvocab * dim // _W, _W)
    out_phys = run(inputs.T, table4)
    return out_phys.transpose(2, 0, 1)


# speedup vs baseline: 1.5781x; 1.3847x over previous
import functools

import jax
import jax.numpy as jnp
from jax import lax
from jax.experimental import pallas as pl
from jax.experimental.pallas import tpu as pltpu
from jax.experimental.pallas import tpu_sc as plsc

_NC = 2
_NS = 16
_NW = _NC * _NS

_D = 32
_G = 128
_W = 128
_TP = 128


def _gather_body(idxT_hbm, table4_hbm, out_hbm, idx_v, idx4_v,
                 r0, r1, t0, t1, sg0, sg1, so0, so1):
    H = idxT_hbm.shape[0]
    B = idxT_hbm.shape[1]
    bw = B // _NW
    jb = bw // _G
    nblk = H * jb
    wid = lax.axis_index("s") * _NC + lax.axis_index("c")
    bbase = wid * bw

    for j in range(jb):
        pltpu.sync_copy(idxT_hbm.at[:, pl.ds(bbase + j * _G, _G)],
                        idx_v.at[j, pl.ds(0, H)])

    def sbody(k, carry):
        j = k // (H * 8)
        rem = k % (H * 8)
        h = rem // 8
        g = rem % 8
        v = idx_v[j, h, pl.ds(g * 16, 16)]
        idx4_v[j, h, pl.ds(g * 16, 16)] = jax.lax.shift_right_logical(v, 2)
        return carry

    lax.fori_loop(0, jb * H * 8, sbody, 0, unroll=8)

    rows = (r0, r1)
    trs = (t0, t1)
    sg = (sg0, sg1)
    so = (so0, so1)
    iota = lax.iota(jnp.int32, 16)
    bidxs = [iota + 16 * g for g in range(8)]

    def issue_gather(i, p):
        pltpu.async_copy(table4_hbm.at[idx4_v.at[i % jb, i // jb]],
                         rows[p], sg[p])

    def drain_gather(p):
        pltpu.make_async_copy(table4_hbm.at[pl.ds(0, _G)], rows[p],
                              sg[p]).wait()

    def transpose(i, p):
        r, t = rows[p], trs[p]
        jj = i % jb
        hh = i // jb
        mcols = [(idx_v[jj, hh, pl.ds(g * 16, 16)] & 3) * 32
                 for g in range(8)]

        def tbody(d, carry):
            dvec = (d + iota) & 31
            for g in range(8):
                v = plsc.load_gather(r, [bidxs[g], mcols[g] + dvec])
                plsc.store_scatter(t, [dvec, bidxs[g]], v)
            return carry

        lax.fori_loop(0, _D, tbody, 0, unroll=2)

    def issue_store(i, p):
        pltpu.async_copy(
            trs[p].at[:, pl.ds(0, _G)],
            out_hbm.at[i // jb, :, pl.ds(bbase + (i % jb) * _G, _G)],
            so[p])

    def drain_store(p):
        pltpu.make_async_copy(trs[p].at[:, pl.ds(0, _G)],
                              out_hbm.at[0, :, pl.ds(0, _G)], so[p]).wait()

    issue_gather(0, 0)
    issue_gather(1, 1)
    drain_gather(0)
    transpose(0, 0)
    issue_store(0, 0)

    def body(g, carry):
        i0 = 2 * g
        drain_store(0)
        issue_gather(i0, 0)
        drain_gather(1)
        transpose(i0 - 1, 1)
        issue_store(i0 - 1, 1)
        drain_store(1)
        issue_gather(i0 + 1, 1)
        drain_gather(0)
        transpose(i0, 0)
        issue_store(i0, 0)
        return carry

    lax.fori_loop(1, nblk // 2, body, 0)

    drain_gather(1)
    transpose(nblk - 1, 1)
    issue_store(nblk - 1, 1)
    drain_store(0)
    drain_store(1)


def _trans_body(tT_hbm, out4_hbm, vi0, vi1, vo0, vo1, si0, si1, so0, so1):
    V = tT_hbm.shape[1]
    nt = V // _G
    wid = lax.axis_index("s") * _NC + lax.axis_index("c")
    vin = (vi0, vi1)
    vout = (vo0, vo1)
    si = (si0, si1)
    so = (so0, so1)
    iota = lax.iota(jnp.int32, 16)
    cvecs = [iota + 16 * g for g in range(8)]
    i0s = [lax.shift_right_logical(c, 2) for c in cvecs]
    m32s = [(c & 3) * 32 for c in cvecs]

    nk = nt // _NW + 1

    def issue_in(t, p):
        pltpu.async_copy(tT_hbm.at[:, pl.ds(t * _G, _G)], vin[p], si[p])

    def drain_in(p):
        pltpu.make_async_copy(tT_hbm.at[:, pl.ds(0, _G)], vin[p],
                              si[p]).wait()

    def transpose(p):
        vm, vo = vin[p], vout[p]

        def tbody(d, carry):
            dvec = (d + iota) & 31
            for g in range(8):
                v = plsc.load_gather(vm, [dvec, cvecs[g]])
                plsc.store_scatter(vo, [i0s[g], m32s[g] + dvec], v)
            return carry

        lax.fori_loop(0, _D, tbody, 0)

    def issue_out(t, p):
        pltpu.async_copy(vout[p], out4_hbm.at[pl.ds(t * _D, _D)], so[p])

    def drain_out(p):
        pltpu.make_async_copy(vout[p], out4_hbm.at[pl.ds(0, _D)],
                              so[p]).wait()

    def step(k, p, first=False):
        t = wid + k * _NW
        tn = t + _NW

        @pl.when(tn < nt)
        def _():
            issue_in(tn, 1 - p)

        @pl.when(t < nt)
        def _():
            drain_in(p)
            if not first:
                drain_out(p)
            transpose(p)
            issue_out(t, p)

    issue_in(wid, 0)
    step(0, 0, first=True)
    step(1, 1, first=True)

    def body(g, carry):
        step(2 * g, 0)
        step(2 * g + 1, 1)
        return carry

    lax.fori_loop(1, nk // 2 + 1, body, 0)
    drain_out(0)
    drain_out(1)


def kernel(inputs, table):
    batch, hist = inputs.shape
    nvocab, dim = table.shape
    bw = batch // _NW
    jb = bw // _G

    mesh = plsc.VectorSubcoreMesh(core_axis_name="c", subcore_axis_name="s")
    run = functools.partial(
        pl.kernel,
        mesh=mesh,
        compiler_params=pltpu.CompilerParams(
            use_tc_tiling_on_sc=True, needs_layout_passes=False),
        out_type=jax.ShapeDtypeStruct((hist, _D, batch), jnp.float32),
        scratch_types=(
            [pltpu.VMEM((jb, (hist + 7) // 8 * 8, _G), jnp.int32)
             for _ in range(2)]
            + [pltpu.VMEM((_G, _W), jnp.float32) for _ in range(2)]
            + [pltpu.VMEM((_D, _TP), jnp.float32) for _ in range(2)]
            + [pltpu.SemaphoreType.DMA for _ in range(4)]
        ),
    )(_gather_body)

    run_t = functools.partial(
        pl.kernel,
        mesh=mesh,
        compiler_params=pltpu.CompilerParams(
            use_tc_tiling_on_sc=True, needs_layout_passes=False),
        out_type=jax.ShapeDtypeStruct((nvocab * dim // _W, _W), jnp.float32),
        scratch_types=(
            [pltpu.VMEM((_D, _G), jnp.float32) for _ in range(4)]
            + [pltpu.SemaphoreType.DMA for _ in range(4)]
        ),
    )(_trans_body)

    table4 = run_t(table.T)
    ntail = nvocab % _G
    if ntail:
        tail4 = table[nvocab - ntail:].reshape(ntail * dim // _W, _W)
        table4 = jax.lax.dynamic_update_slice(
            table4, tail4, ((nvocab - ntail) * dim // _W, 0))
    out_phys = run(inputs.T, table4)
    return out_phys.transpose(2, 0, 1)


# speedup vs baseline: 1.6050x; 1.0170x over previous
import functools

import jax
import jax.numpy as jnp
from jax import lax
from jax.experimental import pallas as pl
from jax.experimental.pallas import tpu as pltpu
from jax.experimental.pallas import tpu_sc as plsc

_NC = 2
_NS = 16
_NW = _NC * _NS

_D = 32
_G = 128
_W = 128
_TP = 128


def _gather_body(idxT_hbm, table4_hbm, out_hbm, idx_v, idx4_v,
                 r0, r1, t0, t1, sg0, sg1, so0, so1):
    H = idxT_hbm.shape[0]
    B = idxT_hbm.shape[1]
    bw = B // _NW
    jb = bw // _G
    nblk = H * jb
    wid = lax.axis_index("s") * _NC + lax.axis_index("c")
    bbase = wid * bw

    for j in range(jb):
        pltpu.sync_copy(idxT_hbm.at[:, pl.ds(bbase + j * _G, _G)],
                        idx_v.at[j, pl.ds(0, H)])

    def sbody(k, carry):
        j = k // (H * 8)
        rem = k % (H * 8)
        h = rem // 8
        g = rem % 8
        v = idx_v[j, h, pl.ds(g * 16, 16)]
        idx4_v[j, h, pl.ds(g * 16, 16)] = jax.lax.shift_right_logical(v, 2)
        return carry

    lax.fori_loop(0, jb * H * 8, sbody, 0, unroll=8)

    rows = (r0, r1)
    trs = (t0, t1)
    sg = (sg0, sg1)
    so = (so0, so1)
    iota = lax.iota(jnp.int32, 16)
    bidxs = [iota + 16 * g for g in range(8)]

    def issue_gather(i, p):
        pltpu.async_copy(table4_hbm.at[idx4_v.at[i % jb, i // jb]],
                         rows[p], sg[p])

    def drain_gather(p):
        pltpu.make_async_copy(table4_hbm.at[pl.ds(0, _G)], rows[p],
                              sg[p]).wait()

    def transpose(i, p):
        r, t = rows[p], trs[p]
        jj = i % jb
        hh = i // jb
        mcols = [(idx_v[jj, hh, pl.ds(g * 16, 16)] & 3) * 32
                 for g in range(8)]

        def tbody(d, carry):
            dvec = (d + iota) & 31
            for g in range(8):
                v = plsc.load_gather(r, [bidxs[g], mcols[g] + dvec])
                plsc.store_scatter(t, [dvec, bidxs[g]], v)
            return carry

        lax.fori_loop(0, _D, tbody, 0, unroll=2)

    def issue_store(i, p):
        pltpu.async_copy(
            trs[p].at[:, pl.ds(0, _G)],
            out_hbm.at[i // jb, :, pl.ds(bbase + (i % jb) * _G, _G)],
            so[p])

    def drain_store(p):
        pltpu.make_async_copy(trs[p].at[:, pl.ds(0, _G)],
                              out_hbm.at[0, :, pl.ds(0, _G)], so[p]).wait()

    issue_gather(0, 0)
    issue_gather(1, 1)
    drain_gather(0)
    transpose(0, 0)
    issue_store(0, 0)

    def body(g, carry):
        i0 = 2 * g
        drain_store(0)
        issue_gather(i0, 0)
        drain_gather(1)
        transpose(i0 - 1, 1)
        issue_store(i0 - 1, 1)
        drain_store(1)
        issue_gather(i0 + 1, 1)
        drain_gather(0)
        transpose(i0, 0)
        issue_store(i0, 0)
        return carry

    lax.fori_loop(1, nblk // 2, body, 0)

    drain_gather(1)
    transpose(nblk - 1, 1)
    issue_store(nblk - 1, 1)
    drain_store(0)
    drain_store(1)


def _trans_body(tT_hbm, out4_hbm, vi0, vi1, vo0, vo1, si0, si1, so0, so1):
    V = tT_hbm.shape[1]
    nt = V // _G
    wid = lax.axis_index("s") * _NC + lax.axis_index("c")
    vin = (vi0, vi1)
    vout = (vo0, vo1)
    si = (si0, si1)
    so = (so0, so1)
    iota = lax.iota(jnp.int32, 16)
    cvecs = [iota + 16 * g for g in range(8)]
    i0s = [lax.shift_right_logical(c, 2) for c in cvecs]
    m32s = [(c & 3) * 32 for c in cvecs]

    nk = nt // _NW + 1

    def issue_in(t, p):
        pltpu.async_copy(tT_hbm.at[:, pl.ds(t * _G, _G)], vin[p], si[p])

    def drain_in(p):
        pltpu.make_async_copy(tT_hbm.at[:, pl.ds(0, _G)], vin[p],
                              si[p]).wait()

    def transpose(p):
        vm, vo = vin[p], vout[p]

        def tbody(d, carry):
            dvec = (d + iota) & 31
            for g in range(8):
                v = plsc.load_gather(vm, [dvec, cvecs[g]])
                plsc.store_scatter(vo, [i0s[g], m32s[g] + dvec], v)
            return carry

        lax.fori_loop(0, _D, tbody, 0, unroll=4)

    def issue_out(t, p):
        pltpu.async_copy(vout[p], out4_hbm.at[pl.ds(t * _D, _D)], so[p])

    def drain_out(p):
        pltpu.make_async_copy(vout[p], out4_hbm.at[pl.ds(0, _D)],
                              so[p]).wait()

    def step(k, p, first=False):
        t = wid + k * _NW
        tn = t + _NW

        @pl.when(tn < nt)
        def _():
            issue_in(tn, 1 - p)

        @pl.when(t < nt)
        def _():
            drain_in(p)
            if not first:
                drain_out(p)
            transpose(p)
            issue_out(t, p)

    issue_in(wid, 0)
    step(0, 0, first=True)
    step(1, 1, first=True)

    def body(g, carry):
        step(2 * g, 0)
        step(2 * g + 1, 1)
        return carry

    lax.fori_loop(1, nk // 2 + 1, body, 0)
    drain_out(0)
    drain_out(1)


def kernel(inputs, table):
    batch, hist = inputs.shape
    nvocab, dim = table.shape
    bw = batch // _NW
    jb = bw // _G

    mesh = plsc.VectorSubcoreMesh(core_axis_name="c", subcore_axis_name="s")
    run = functools.partial(
        pl.kernel,
        mesh=mesh,
        compiler_params=pltpu.CompilerParams(
            use_tc_tiling_on_sc=True, needs_layout_passes=False),
        out_type=jax.ShapeDtypeStruct((hist, _D, batch), jnp.float32),
        scratch_types=(
            [pltpu.VMEM((jb, (hist + 7) // 8 * 8, _G), jnp.int32)
             for _ in range(2)]
            + [pltpu.VMEM((_G, _W), jnp.float32) for _ in range(2)]
            + [pltpu.VMEM((_D, _TP), jnp.float32) for _ in range(2)]
            + [pltpu.SemaphoreType.DMA for _ in range(4)]
        ),
    )(_gather_body)

    run_t = functools.partial(
        pl.kernel,
        mesh=mesh,
        compiler_params=pltpu.CompilerParams(
            use_tc_tiling_on_sc=True, needs_layout_passes=False),
        out_type=jax.ShapeDtypeStruct((nvocab * dim // _W, _W), jnp.float32),
        scratch_types=(
            [pltpu.VMEM((_D, _G), jnp.float32) for _ in range(4)]
            + [pltpu.SemaphoreType.DMA for _ in range(4)]
        ),
    )(_trans_body)

    table4 = run_t(table.T)
    ntail = nvocab % _G
    if ntail:
        tail4 = table[nvocab - ntail:].reshape(ntail * dim // _W, _W)
        table4 = jax.lax.dynamic_update_slice(
            table4, tail4, ((nvocab - ntail) * dim // _W, 0))
    out_phys = run(inputs.T, table4)
    return out_phys.transpose(2, 0, 1)
